# baseline (speedup 1.0000x reference)
import jax
import jax.numpy as jnp
from jax.experimental import pallas as pl
from jax.experimental.pallas import tpu as pltpu

C_M, C_Z = 256, 128
MIN_BIN, MAX_BIN, NO_BINS = 3.25, 20.75, 15
INF = 1e8
EPS = 1e-5
N_RES = 768

BI = 64
BJ = 128
CH = 8


def _fused_kernel(z_ref, xi_ref, xjt_ref, m_ref, w_ref,
                  gz_ref, bias_ref, gm_ref, bm_ref, sq_ref, up_ref,
                  zo_ref, mo_ref):
    j = pl.program_id(1)

    @pl.when(j == 0)
    def _():
        mt = m_ref[...]
        mu = jnp.mean(mt, axis=-1, keepdims=True)
        dm = mt - mu
        var = jnp.mean(dm * dm, axis=-1, keepdims=True)
        mo_ref[...] = dm * jax.lax.rsqrt(var + EPS) * gm_ref[...] + bm_ref[...]

    gz = gz_ref[...]
    bias = bias_ref[...]
    sq = sq_ref[...]
    up = up_ref[...]
    w = w_ref[...]
    xjt = xjt_ref[...]
    inv = jnp.float32(1.0 / C_Z)

    def chunk(c, carry):
        r0 = pl.multiple_of(c * CH, CH)
        xi = xi_ref[pl.ds(r0, CH)]
        d0 = xi[:, 0:1] - xjt[0:1, :]
        d1 = xi[:, 1:2] - xjt[1:2, :]
        dc = xi[:, 2:3] - xjt[2:3, :]
        d2 = d0 * d0 + d1 * d1 + dc * dc

        zt = z_ref[pl.ds(r0, CH)]
        s1 = jnp.sum(zt, axis=-1, keepdims=True)
        s2 = jnp.sum(zt * zt, axis=-1, keepdims=True)
        mu = s1 * inv
        var = s2 * inv - mu * mu
        rs = jax.lax.rsqrt(var + EPS)
        ln = (zt - mu) * rs * gz + bias

        d2e = d2[:, :, None]
        oh = jnp.where(d2e > sq,
                       jnp.where(d2e < up, 1.0, 0.0),
                       0.0)
        ohb = oh.astype(jnp.bfloat16).reshape(CH * BJ, 16)
        demb = jnp.dot(ohb, w, preferred_element_type=jnp.float32)
        zo_ref[pl.ds(r0, CH)] = ln + demb.reshape(CH, BJ, C_Z)
        return carry

    jax.lax.fori_loop(0, BI // CH, chunk, 0)


def kernel(m, z, x, w_lin, b_lin, g_m, be_m, g_z, be_z):
    f32 = jnp.float32
    m2 = m[0]
    z3 = z[0]
    x2 = x[0]
    xjt = x2.T

    bins = jnp.linspace(MIN_BIN, MAX_BIN, NO_BINS, dtype=f32)
    sq = bins ** 2
    upper = jnp.concatenate([sq[1:], jnp.array([INF], dtype=f32)])
    inf1 = jnp.array([INF], dtype=f32)
    sq16 = jnp.concatenate([sq, inf1]).reshape(1, 1, 16)
    up16 = jnp.concatenate([upper, inf1]).reshape(1, 1, 16)
    w16 = jnp.concatenate(
        [w_lin.T, jnp.zeros((1, C_Z), f32)], axis=0).astype(jnp.bfloat16)

    bias3 = (be_z + b_lin).reshape(1, 1, C_Z)
    gz3 = g_z.reshape(1, 1, C_Z)
    gm2 = g_m.reshape(1, C_M)
    bm2 = be_m.reshape(1, C_M)

    grid = (N_RES // BI, N_RES // BJ)
    zo, mo = pl.pallas_call(
        _fused_kernel,
        grid=grid,
        in_specs=[
            pl.BlockSpec((BI, BJ, C_Z), lambda i, j: (i, j, 0)),
            pl.BlockSpec((BI, 3), lambda i, j: (i, 0)),
            pl.BlockSpec((3, BJ), lambda i, j: (0, j)),
            pl.BlockSpec((BI, C_M), lambda i, j: (i, 0)),
            pl.BlockSpec((16, C_Z), lambda i, j: (0, 0)),
            pl.BlockSpec((1, 1, C_Z), lambda i, j: (0, 0, 0)),
            pl.BlockSpec((1, 1, C_Z), lambda i, j: (0, 0, 0)),
            pl.BlockSpec((1, C_M), lambda i, j: (0, 0)),
            pl.BlockSpec((1, C_M), lambda i, j: (0, 0)),
            pl.BlockSpec((1, 1, 16), lambda i, j: (0, 0, 0)),
            pl.BlockSpec((1, 1, 16), lambda i, j: (0, 0, 0)),
        ],
        out_specs=[
            pl.BlockSpec((BI, BJ, C_Z), lambda i, j: (i, j, 0)),
            pl.BlockSpec((BI, C_M), lambda i, j: (i, 0)),
        ],
        out_shape=[
            jax.ShapeDtypeStruct((N_RES, N_RES, C_Z), f32),
            jax.ShapeDtypeStruct((N_RES, C_M), f32),
        ],
        compiler_params=pltpu.CompilerParams(
            dimension_semantics=("parallel", "arbitrary"),
        ),
        name="recycling_embedder_fused",
    )(z3, x2, xjt, m2, w16, gz3, bias3, gm2, bm2, sq16, up16)

    return mo[None], zo[None]


# speedup vs baseline: 1.7706x; 1.7706x over previous
import jax
import jax.numpy as jnp
from jax.experimental import pallas as pl
from jax.experimental.pallas import tpu as pltpu

C_M, C_Z = 256, 128
MIN_BIN, MAX_BIN, NO_BINS = 3.25, 20.75, 15
INF = 1e8
EPS = 1e-5
N_RES = 768

BI = 192
BJ = 128
CH = 4


def _fused_kernel(z_ref, xi_ref, xjt_ref, m_ref, w_ref, gz_ref,
                  gm_ref, bm_ref, sq_ref,
                  zo_ref, mo_ref):
    j = pl.program_id(1)

    @pl.when(j == 0)
    def _():
        mt = m_ref[...]
        mu = jnp.mean(mt, axis=-1, keepdims=True)
        dm = mt - mu
        var = jnp.mean(dm * dm, axis=-1, keepdims=True)
        mo_ref[...] = dm * jax.lax.rsqrt(var + EPS) * gm_ref[...] + bm_ref[...]

    gz = gz_ref[...]
    sq = sq_ref[...]
    w = w_ref[...]
    xjt = xjt_ref[...]
    onesd = jnp.full((C_Z, C_Z), 1.0 / C_Z, jnp.float32)

    xi = xi_ref[...]
    d0 = xi[:, 0:1] - xjt[0:1, :]
    d1 = xi[:, 1:2] - xjt[1:2, :]
    dc = xi[:, 2:3] - xjt[2:3, :]
    d2f = d0 * d0 + d1 * d1 + dc * dc

    for bi in range(0, BI, CH):
        d2 = d2f[bi:bi + CH]
        zt2 = z_ref[bi:bi + CH].reshape(CH * BJ, C_Z)
        mu = jnp.dot(zt2, onesd, preferred_element_type=jnp.float32)
        e2 = jnp.dot(zt2 * zt2, onesd, preferred_element_type=jnp.float32)
        rs = jax.lax.rsqrt(e2 - mu * mu + EPS)
        ln = (zt2 - mu) * rs * gz

        d2e = d2[:, :, None]
        oh = jnp.where(d2e > sq, 1.0, 0.0)
        ohb = oh.astype(jnp.bfloat16).reshape(CH * BJ, 16)
        demb = jnp.dot(ohb, w, preferred_element_type=jnp.float32)
        zo_ref[bi:bi + CH] = (ln + demb).reshape(CH, BJ, C_Z)


def kernel(m, z, x, w_lin, b_lin, g_m, be_m, g_z, be_z):
    f32 = jnp.float32
    m2 = m[0]
    z3 = z[0]
    x2 = x[0]
    xjt = x2.T

    bins = jnp.linspace(MIN_BIN, MAX_BIN, NO_BINS, dtype=f32)
    sq = bins ** 2
    sq16 = jnp.concatenate(
        [sq, jnp.array([-INF], dtype=f32)]).reshape(1, 1, 16)
    wt = w_lin.T
    vrows = []
    acc = jnp.zeros((C_Z,), f32)
    for k in range(NO_BINS):
        v = (wt[k] - acc).astype(jnp.bfloat16)
        vrows.append(v[None, :])
        acc = acc + v.astype(f32)
    vrows.append((be_z + b_lin)[None, :].astype(jnp.bfloat16))
    w16 = jnp.concatenate(vrows, axis=0)

    gz2 = g_z.reshape(1, C_Z)
    gm2 = g_m.reshape(1, C_M)
    bm2 = be_m.reshape(1, C_M)

    grid = (N_RES // BI, N_RES // BJ)
    zo, mo = pl.pallas_call(
        _fused_kernel,
        grid=grid,
        in_specs=[
            pl.BlockSpec((BI, BJ, C_Z), lambda i, j: (i, j, 0)),
            pl.BlockSpec((BI, 3), lambda i, j: (i, 0)),
            pl.BlockSpec((3, BJ), lambda i, j: (0, j)),
            pl.BlockSpec((BI, C_M), lambda i, j: (i, 0)),
            pl.BlockSpec((16, C_Z), lambda i, j: (0, 0)),
            pl.BlockSpec((1, C_Z), lambda i, j: (0, 0)),
            pl.BlockSpec((1, C_M), lambda i, j: (0, 0)),
            pl.BlockSpec((1, C_M), lambda i, j: (0, 0)),
            pl.BlockSpec((1, 1, 16), lambda i, j: (0, 0, 0)),
        ],
        out_specs=[
            pl.BlockSpec((BI, BJ, C_Z), lambda i, j: (i, j, 0)),
            pl.BlockSpec((BI, C_M), lambda i, j: (i, 0)),
        ],
        out_shape=[
            jax.ShapeDtypeStruct((N_RES, N_RES, C_Z), f32),
            jax.ShapeDtypeStruct((N_RES, C_M), f32),
        ],
        compiler_params=pltpu.CompilerParams(
            dimension_semantics=("parallel", "arbitrary"),
            vmem_limit_bytes=56 * 1024 * 1024,
        ),
        name="recycling_embedder_fused",
    )(z3, x2, xjt, m2, w16, gz2, gm2, bm2, sq16)

    return mo[None], zo[None]
